# baseline (speedup 1.0000x reference)
import functools

import jax
import jax.numpy as jnp
from jax import lax
from jax.experimental import pallas as pl
from jax.experimental.pallas import tpu as pltpu
from jax.experimental.pallas import tpu_sc as plsc

BATCH = 16384
D = 64
NUM_CORES = 2
NUM_SUBCORES = 16
NUM_WORKERS = NUM_CORES * NUM_SUBCORES
B_PER_W = BATCH // NUM_WORKERS
CHUNK = 128
N_CHUNKS = B_PER_W // CHUNK
LANES = 16

_mesh = plsc.VectorSubcoreMesh(core_axis_name="c", subcore_axis_name="s")


@functools.partial(
    pl.kernel,
    mesh=_mesh,
    out_type=jax.ShapeDtypeStruct((BATCH,), jnp.float32),
    scratch_types=[
        pltpu.VMEM((N_CHUNKS, CHUNK), jnp.int32),
        pltpu.VMEM((N_CHUNKS, CHUNK), jnp.int32),
        pltpu.VMEM((B_PER_W, D), jnp.float32),
        pltpu.VMEM((B_PER_W, D), jnp.float32),
        pltpu.VMEM((B_PER_W,), jnp.float32),
        pltpu.SemaphoreType.DMA,
    ],
    compiler_params=pltpu.CompilerParams(
        needs_layout_passes=False, use_tc_tiling_on_sc=False),
)
def _sc_dot(uidx_hbm, midx_hbm, utab_hbm, mtab_hbm, out_hbm,
            uidx_v, midx_v, urows_v, mrows_v, out_v, sem):
    wid = lax.axis_index("s") * NUM_CORES + lax.axis_index("c")
    base = wid * B_PER_W

    pltpu.sync_copy(uidx_hbm.at[wid], uidx_v)
    pltpu.sync_copy(midx_hbm.at[wid], midx_v)

    copies = []
    for j in range(N_CHUNKS):
        copies.append(pltpu.async_copy(
            utab_hbm.at[uidx_v.at[j]],
            urows_v.at[pl.ds(j * CHUNK, CHUNK)], sem))
        copies.append(pltpu.async_copy(
            mtab_hbm.at[midx_v.at[j]],
            mrows_v.at[pl.ds(j * CHUNK, CHUNK)], sem))
    for c in copies:
        c.wait()

    lane_iota = jax.lax.iota(jnp.int32, LANES)

    def body(g, carry):
        bvec = g * LANES + lane_iota
        acc = jnp.zeros((LANES,), jnp.float32)
        for d in range(D):
            dvec = jnp.full((LANES,), d, jnp.int32)
            uv = plsc.load_gather(urows_v, [bvec, dvec])
            mv = plsc.load_gather(mrows_v, [bvec, dvec])
            acc = acc + uv * mv
        out_v[pl.ds(g * LANES, LANES)] = acc
        return carry

    lax.fori_loop(0, B_PER_W // LANES, body, 0)

    pltpu.sync_copy(out_v, out_hbm.at[pl.ds(base, B_PER_W)])


def kernel(userIndices, movieIndices, user_table, movie_table):
    u = userIndices.astype(jnp.int32).reshape(NUM_WORKERS, N_CHUNKS, CHUNK)
    m = movieIndices.astype(jnp.int32).reshape(NUM_WORKERS, N_CHUNKS, CHUNK)
    return _sc_dot(u, m, user_table, movie_table)
